# baseline (speedup 1.0000x reference)
import jax
import jax.numpy as jnp
from jax import lax
from jax.experimental import pallas as pl
from jax.experimental.pallas import tpu as pltpu
from jax.experimental.pallas import tpu_sc as plsc

NV_PREV = 10242
NW = 32
PER_W = NV_PREV // NW
TAIL = NV_PREV - PER_W * NW
NCHUNK = 8
CH = PER_W // NCHUNK
NBUF = 3


def _make_body(num_cores):
    def _sc_body(x_hbm, o_hbm, buf, in_sems, out_sems, tail_sem):
        xr = x_hbm
        orr = o_hbm
        wid = lax.axis_index("s") * num_cores + lax.axis_index("c")
        base = wid * PER_W

        def in_cp(k):
            return pltpu.make_async_copy(
                xr.at[pl.ds(base + k * CH, CH)], buf.at[k % NBUF],
                in_sems.at[k % NBUF])

        def out_cp(k):
            return pltpu.make_async_copy(
                buf.at[k % NBUF], orr.at[pl.ds(base + k * CH, CH)],
                out_sems.at[k % NBUF])

        in_cp(0).start()
        in_cp(1).start()
        for k in range(NCHUNK):
            in_cp(k).wait()
            out_cp(k).start()
            nk = k + 2
            if nk < NCHUNK:
                if nk >= NBUF:
                    out_cp(nk - NBUF).wait()
                in_cp(nk).start()
        for k in range(NCHUNK - NBUF, NCHUNK):
            out_cp(k).wait()

        @pl.when(wid == 0)
        def _tail():
            cp = pltpu.make_async_copy(
                xr.at[pl.ds(NW * PER_W, TAIL)],
                buf.at[0, pl.ds(0, TAIL)], tail_sem)
            cp.start()
            cp.wait()
            cp2 = pltpu.make_async_copy(
                buf.at[0, pl.ds(0, TAIL)],
                orr.at[pl.ds(NW * PER_W, TAIL)], tail_sem)
            cp2.start()
            cp2.wait()

    return _sc_body


def kernel(x):
    n, a, b, c = x.shape
    mesh = plsc.VectorSubcoreMesh(core_axis_name="c", subcore_axis_name="s")
    run = pl.kernel(
        _make_body(mesh.num_cores),
        out_type=jax.ShapeDtypeStruct((NV_PREV, a, b, c), x.dtype),
        mesh=mesh,
        compiler_params=pltpu.CompilerParams(use_tc_tiling_on_sc=False),
        scratch_types=[
            pltpu.VMEM((NBUF, CH, 4, 4, 64), x.dtype),
            pltpu.SemaphoreType.DMA((NBUF,)),
            pltpu.SemaphoreType.DMA((NBUF,)),
            pltpu.SemaphoreType.DMA,
        ],
    )
    return run(x)


# speedup vs baseline: 1.3207x; 1.3207x over previous
import jax, jax.numpy as jnp
from jax.experimental import pallas as pl
from jax.experimental.pallas import tpu as pltpu

NV_PREV = 10242
B = 569

def _body(x_ref, o_ref):
    o_ref[...] = x_ref[...]

def kernel(x):
    n, a, b, c = x.shape
    return pl.pallas_call(
        _body,
        grid=(NV_PREV // B,),
        in_specs=[pl.BlockSpec((B, a, b, c), lambda i: (i, 0, 0, 0))],
        out_specs=pl.BlockSpec((B, a, b, c), lambda i: (i, 0, 0, 0)),
        out_shape=jax.ShapeDtypeStruct((NV_PREV, a, b, c), x.dtype),
    )(x)


# speedup vs baseline: 20.1987x; 15.2944x over previous
import jax, jax.numpy as jnp
from jax import lax
from jax.experimental import pallas as pl
from jax.experimental.pallas import tpu as pltpu

NV_PREV = 10242
BLK = 1024

def _body(x_ref, o_ref):
    o_ref[...] = x_ref[...]

def kernel(x):
    n, a, b, c = x.shape
    xt = lax.transpose(x, (1, 2, 3, 0))
    out_t = pl.pallas_call(
        _body,
        grid=(pl.cdiv(NV_PREV, BLK),),
        in_specs=[pl.BlockSpec((a, b, c, BLK), lambda i: (0, 0, 0, i))],
        out_specs=pl.BlockSpec((a, b, c, BLK), lambda i: (0, 0, 0, i)),
        out_shape=jax.ShapeDtypeStruct((a, b, c, NV_PREV), x.dtype),
    )(xt)
    return lax.transpose(out_t, (3, 0, 1, 2))
